# Initial kernel scaffold; baseline (speedup 1.0000x reference)
import jax
import jax.numpy as jnp
from jax.experimental import pallas as pl


def kernel(embedding, adj_row, adj_col, adj_values, W0, W1):
    raise NotImplementedError("write your pallas kernel here")


# speedup vs baseline: 2.8171x; 2.8171x over previous
import functools

import jax
import jax.numpy as jnp
from jax import lax
from jax.experimental import pallas as pl
from jax.experimental.pallas import tpu as pltpu
from jax.experimental.pallas import tpu_sc as plsc

N = 10000
D = 128
E = 320000

NUM_CORES = 2
NUM_SUBCORES = 16
LANES = 16
EDGE_BLOCK = 128
NB = 80
PE = NUM_CORES * NUM_SUBCORES * NB * EDGE_BLOCK
N_PAD = 10240
ROWS_PER_TILE = N_PAD // NUM_SUBCORES


def _make_sc_aggregate(n_pad, nb, edge_block, d=D, interpret=False):
  rows_per_tile = n_pad // NUM_SUBCORES

  chunk = 8
  nchunks = nb // chunk

  def body(y_hbm, cols_hbm, rows_hbm, vals_hbm, zeros_hbm,
           out_hbm, cols_v, rows_v, vals_v, gath, scaled, acc, sem):
    c = lax.axis_index("c")
    s = lax.axis_index("s")

    pltpu.sync_copy(zeros_hbm, acc.at[pl.ds(s * rows_per_tile, rows_per_tile)])

    plsc.subcore_barrier()

    def chunk_body(g, carry):
        pltpu.sync_copy(cols_hbm.at[c, s, pl.ds(g * chunk, chunk)], cols_v)
        pltpu.sync_copy(rows_hbm.at[c, s, pl.ds(g * chunk, chunk)], rows_v)
        pltpu.sync_copy(
            vals_hbm.at[c, s, pl.ds(g * chunk * edge_block,
                                    chunk * edge_block)], vals_v)

        def block(k, kcarry):
            pltpu.async_copy(y_hbm.at[cols_v.at[k]], gath, sem).wait()

            def edge(e, ecarry):
                fvec = jnp.full((LANES,), k * edge_block + e, jnp.int32)
                val = plsc.load_gather(vals_v, [fvec])
                for j in range(d // LANES):
                    sl = pl.ds(j * LANES, LANES)
                    scaled[e, sl] = gath[e, sl] * val
                return ecarry

            lax.fori_loop(0, edge_block, edge, 0)

            pltpu.sync_copy(scaled, acc.at[rows_v.at[k]], add=True)
            return kcarry

        lax.fori_loop(0, chunk, block, 0)
        return carry

    lax.fori_loop(0, nchunks, chunk_body, 0)

    plsc.subcore_barrier()
    sl = pl.ds(s * rows_per_tile, rows_per_tile)
    pltpu.sync_copy(acc.at[sl], out_hbm.at[c, sl])

  return pl.kernel(
      body,
      out_type=jax.ShapeDtypeStruct((NUM_CORES, n_pad, d), jnp.float32),
      mesh=plsc.VectorSubcoreMesh(
          core_axis_name="c", subcore_axis_name="s",
          num_cores=NUM_CORES, num_subcores=NUM_SUBCORES),
      compiler_params=pltpu.CompilerParams(needs_layout_passes=False),
      interpret=interpret,
      scratch_types=[
          pltpu.VMEM((8, edge_block), jnp.int32),
          pltpu.VMEM((8, edge_block), jnp.int32),
          pltpu.VMEM((8 * edge_block,), jnp.float32),
          pltpu.VMEM((edge_block, d), jnp.float32),
          pltpu.VMEM((edge_block, d), jnp.float32),
          pltpu.VMEM_SHARED((n_pad, d), jnp.float32),
          pltpu.SemaphoreType.DMA,
      ],
  )


_sc_aggregate = _make_sc_aggregate(N_PAD, NB, EDGE_BLOCK)


def _mm_body(x_ref, w_ref, y_ref):
    y_ref[...] = lax.dot_general(
        x_ref[...], w_ref[...], (((1,), (1,)), ((), ())),
        preferred_element_type=jnp.float32)


_mm = pl.pallas_call(
    _mm_body,
    out_shape=jax.ShapeDtypeStruct((N, D), jnp.float32),
)


def _normalize(h):
    norm = jnp.sqrt(jnp.sum(h * h, axis=-1, keepdims=True))
    return h / jnp.maximum(norm, 1e-12)


def _norm_mm_body(p_ref, w_ref, h_ref, y_ref):
    agg = p_ref[0, :N] + p_ref[1, :N]
    h_ref[...] = _normalize(agg)
    y_ref[...] = lax.dot_general(
        agg, w_ref[...], (((1,), (1,)), ((), ())),
        preferred_element_type=jnp.float32)


_norm_mm = pl.pallas_call(
    _norm_mm_body,
    out_shape=(jax.ShapeDtypeStruct((N, D), jnp.float32),
               jax.ShapeDtypeStruct((N, D), jnp.float32)),
)


def _final_body(e_ref, h1_ref, p_ref, o_ref):
    h2 = _normalize(p_ref[0, :N] + p_ref[1, :N])
    o_ref[...] = (e_ref[...] + h1_ref[...] + h2) * (1.0 / 3.0)


_final = pl.pallas_call(
    _final_body,
    out_shape=jax.ShapeDtypeStruct((N, D), jnp.float32),
)


def kernel(embedding, adj_row, adj_col, adj_values, W0, W1):
    pad = PE - E
    cols = jnp.pad(adj_col.astype(jnp.int32), (0, pad)).reshape(
        NUM_CORES, NUM_SUBCORES, NB, EDGE_BLOCK)
    rows = jnp.pad(adj_row.astype(jnp.int32), (0, pad)).reshape(
        NUM_CORES, NUM_SUBCORES, NB, EDGE_BLOCK)
    vals = jnp.pad(adj_values, (0, pad)).reshape(
        NUM_CORES, NUM_SUBCORES, NB * EDGE_BLOCK)
    zeros = jnp.zeros((ROWS_PER_TILE, D), jnp.float32)

    y0 = _mm(embedding, W0)
    p1 = _sc_aggregate(y0, cols, rows, vals, zeros)
    h1, y1 = _norm_mm(p1, W1)
    p2 = _sc_aggregate(y1, cols, rows, vals, zeros)
    return _final(embedding, h1, p2)
